# Initial kernel scaffold; baseline (speedup 1.0000x reference)
import jax
import jax.numpy as jnp
from jax.experimental import pallas as pl


def kernel(x, edge_index, W1, b1, W2, b2):
    raise NotImplementedError("write your pallas kernel here")


# speedup vs baseline: 27.4278x; 27.4278x over previous
import functools

import jax
import jax.numpy as jnp
from jax import lax
from jax.experimental import pallas as pl
from jax.experimental.pallas import tpu as pltpu
from jax.experimental.pallas import tpu_sc as plsc

_NC = 2
_NS = 16
_NW = _NC * _NS
_CH = 128

_f32 = jnp.float32


def _mesh():
    return plsc.VectorSubcoreMesh(core_axis_name="c", subcore_axis_name="s")


def _deg_call(dst3, ones_rows, zinit, *, NP, K):
    F = ones_rows.shape[1]

    @functools.partial(
        pl.kernel,
        mesh=_mesh(),
        compiler_params=pltpu.CompilerParams(use_tc_tiling_on_sc=False),
        out_type=jax.ShapeDtypeStruct((_NC, NP, F), _f32),
        scratch_types=[
            pltpu.VMEM_SHARED((NP, F), _f32),
            pltpu.VMEM((K, _CH), jnp.int32),
            pltpu.VMEM((_CH, F), _f32),
        ],
    )
    def k(dst_h, ones_h, zinit_h, out_h, spmem, idx_d, rows):
        c = lax.axis_index("c")
        s = lax.axis_index("s")
        wid = s * _NC + c
        nps = NP // _NS
        pltpu.sync_copy(zinit_h.at[pl.ds(s * nps, nps)],
                        spmem.at[pl.ds(s * nps, nps)])
        pltpu.sync_copy(dst_h.at[wid], idx_d)
        pltpu.sync_copy(ones_h, rows)
        plsc.subcore_barrier()

        def step(j, _):
            pltpu.sync_copy(rows, spmem.at[idx_d.at[j]], add=True)
            return ()

        lax.fori_loop(0, K, step, (), unroll=4)
        plsc.subcore_barrier()
        nps = NP // _NS
        pltpu.sync_copy(spmem.at[pl.ds(s * nps, nps)],
                        out_h.at[c].at[pl.ds(s * nps, nps)])

    return k(dst3, ones_rows, zinit)


def _agg_call(table, src3, dst3, zinit, *, NP, K):
    F = table.shape[1]

    @functools.partial(
        pl.kernel,
        mesh=_mesh(),
        compiler_params=pltpu.CompilerParams(use_tc_tiling_on_sc=False),
        out_type=jax.ShapeDtypeStruct((_NC, NP, F), _f32),
        scratch_types=[
            pltpu.VMEM_SHARED((NP, F), _f32),
            pltpu.VMEM((K, _CH), jnp.int32),
            pltpu.VMEM((K, _CH), jnp.int32),
            pltpu.VMEM((2, _CH, F), _f32),
            pltpu.SemaphoreType.DMA,
            pltpu.SemaphoreType.DMA,
        ],
    )
    def k(table_h, src_h, dst_h, zinit_h, out_h, spmem, idx_s, idx_d, rows,
          sem0, sem1):
        c = lax.axis_index("c")
        s = lax.axis_index("s")
        wid = s * _NC + c
        nps = NP // _NS
        pltpu.sync_copy(zinit_h.at[pl.ds(s * nps, nps)],
                        spmem.at[pl.ds(s * nps, nps)])
        pltpu.sync_copy(src_h.at[wid], idx_s)
        pltpu.sync_copy(dst_h.at[wid], idx_d)
        plsc.subcore_barrier()

        pltpu.make_async_copy(table_h.at[idx_s.at[0]], rows.at[0], sem0).start()

        def step(i, _):
            j0 = i * 2
            pltpu.make_async_copy(table_h.at[idx_s.at[j0 + 1]], rows.at[1],
                                  sem1).start()
            pltpu.make_async_copy(table_h.at[idx_s.at[j0]], rows.at[0],
                                  sem0).wait()
            pltpu.sync_copy(rows.at[0], spmem.at[idx_d.at[j0]], add=True)

            @pl.when(j0 + 2 < K)
            def _():
                pltpu.make_async_copy(table_h.at[idx_s.at[j0 + 2]], rows.at[0],
                                      sem0).start()

            pltpu.make_async_copy(table_h.at[idx_s.at[j0 + 1]], rows.at[1],
                                  sem1).wait()
            pltpu.sync_copy(rows.at[1], spmem.at[idx_d.at[j0 + 1]], add=True)
            return ()

        lax.fori_loop(0, K // 2, step, ())
        plsc.subcore_barrier()
        nps = NP // _NS
        pltpu.sync_copy(spmem.at[pl.ds(s * nps, nps)],
                        out_h.at[c].at[pl.ds(s * nps, nps)])

    return k(table, src3, dst3, zinit)


def _tc_a(degp, x, w1):
    N = x.shape[0]
    H = w1.shape[1]

    def body(degp_ref, x_ref, w1_ref, hs1_ref, dinv_ref):
        deg = degp_ref[0, 0:N, 0:1] + degp_ref[1, 0:N, 0:1] + 1.0
        dinv = lax.rsqrt(deg)
        h = jnp.dot(x_ref[...], w1_ref[...], preferred_element_type=_f32)
        hs1_ref[...] = h * dinv
        dinv_ref[...] = dinv

    return pl.pallas_call(
        body,
        out_shape=(jax.ShapeDtypeStruct((N, H), _f32),
                   jax.ShapeDtypeStruct((N, 1), _f32)),
    )(degp, x, w1)


def _tc_b(m1, hs1, dinv, b1r, w2p):
    N = hs1.shape[0]
    F2 = w2p.shape[1]

    def body(m1_ref, hs1_ref, dinv_ref, b1_ref, w2_ref, hs2_ref):
        dinv = dinv_ref[...]
        z = dinv * (m1_ref[0, 0:N] + m1_ref[1, 0:N] + hs1_ref[...]) + b1_ref[...]
        z = jnp.maximum(z, 0.0)
        h2 = jnp.dot(z, w2_ref[...], preferred_element_type=_f32)
        hs2_ref[...] = h2 * dinv

    return pl.pallas_call(
        body,
        out_shape=jax.ShapeDtypeStruct((N, F2), _f32),
    )(m1, hs1, dinv, b1r, w2p)


def _tc_c(m2, hs2, dinv, b2r):
    N, F2 = hs2.shape

    def body(m2_ref, hs2_ref, dinv_ref, b2_ref, out_ref):
        out_ref[...] = (dinv_ref[...]
                        * (m2_ref[0, 0:N] + m2_ref[1, 0:N] + hs2_ref[...])
                        + b2_ref[...])

    return pl.pallas_call(
        body,
        out_shape=jax.ShapeDtypeStruct((N, F2), _f32),
    )(m2, hs2, dinv, b2r)


def kernel(x, edge_index, W1, b1, W2, b2):
    N, D = x.shape
    H = W1.shape[1]
    C = W2.shape[1]
    E = edge_index.shape[1]

    F2 = 48
    NP = -(-(N + 1) // 128) * 128
    per_w = -(-E // (_NW * 2 * _CH)) * (2 * _CH)
    K = per_w // _CH
    E_pad = per_w * _NW
    pad = E_pad - E

    src = edge_index[0]
    dst = edge_index[1]
    src3 = jnp.concatenate([src, jnp.zeros((pad,), jnp.int32)]).reshape(
        _NW, K, _CH)
    dst3 = jnp.concatenate([dst, jnp.full((pad,), N, jnp.int32)]).reshape(
        _NW, K, _CH)
    zeros16 = jnp.zeros((NP, H), _f32)
    zeros48 = jnp.zeros((NP, F2), _f32)
    ones16 = jnp.ones((_CH, H), _f32)
    w2p = jnp.pad(W2, ((0, 0), (0, F2 - C)))
    b1r = b1.reshape(1, H)
    b2r = jnp.pad(b2, (0, F2 - C)).reshape(1, F2)

    degp = _deg_call(dst3, ones16, zeros16, NP=NP, K=K)
    hs1, dinv = _tc_a(degp, x, W1)
    m1 = _agg_call(hs1, src3, dst3, zeros16, NP=NP, K=K)
    hs2 = _tc_b(m1, hs1, dinv, b1r, w2p)
    m2 = _agg_call(hs2, src3, dst3, zeros48, NP=NP, K=K)
    out48 = _tc_c(m2, hs2, dinv, b2r)
    return out48[:, :C]


# speedup vs baseline: 54.6712x; 1.9933x over previous
import functools

import jax
import jax.numpy as jnp
from jax import lax
from jax.experimental import pallas as pl
from jax.experimental.pallas import tpu as pltpu
from jax.experimental.pallas import tpu_sc as plsc

_NC = 2
_NS = 16
_NW = _NC * _NS
_CH = 128
_NBUF = 8

_f32 = jnp.float32


def _mesh():
    return plsc.VectorSubcoreMesh(core_axis_name="c", subcore_axis_name="s")


def _deg_call(dst3, ones_rows, zinit, *, NP, K):
    F = ones_rows.shape[1]

    @functools.partial(
        pl.kernel,
        mesh=_mesh(),
        compiler_params=pltpu.CompilerParams(use_tc_tiling_on_sc=False),
        out_type=jax.ShapeDtypeStruct((_NC, NP, F), _f32),
        scratch_types=[
            pltpu.VMEM_SHARED((NP, F), _f32),
            pltpu.VMEM((K, _CH), jnp.int32),
            pltpu.VMEM((_CH, F), _f32),
        ],
    )
    def k(dst_h, ones_h, zinit_h, out_h, spmem, idx_d, rows):
        c = lax.axis_index("c")
        s = lax.axis_index("s")
        wid = s * _NC + c
        nps = NP // _NS
        pltpu.sync_copy(zinit_h.at[pl.ds(s * nps, nps)],
                        spmem.at[pl.ds(s * nps, nps)])
        pltpu.sync_copy(dst_h.at[wid], idx_d)
        pltpu.sync_copy(ones_h, rows)
        plsc.subcore_barrier()

        def step(j, _):
            pltpu.sync_copy(rows, spmem.at[idx_d.at[j]], add=True)
            return ()

        lax.fori_loop(0, K, step, (), unroll=4)
        plsc.subcore_barrier()
        nps = NP // _NS
        pltpu.sync_copy(spmem.at[pl.ds(s * nps, nps)],
                        out_h.at[c].at[pl.ds(s * nps, nps)])

    return k(dst3, ones_rows, zinit)


def _agg_call(table, src3, dst3, zinit, *, NP, K):
    F = table.shape[1]

    @functools.partial(
        pl.kernel,
        mesh=_mesh(),
        compiler_params=pltpu.CompilerParams(use_tc_tiling_on_sc=False),
        out_type=jax.ShapeDtypeStruct((_NC, NP, F), _f32),
        scratch_types=[
            pltpu.VMEM_SHARED((NP, F), _f32),
            pltpu.VMEM((K, _CH), jnp.int32),
            pltpu.VMEM((K, _CH), jnp.int32),
            pltpu.VMEM((_NBUF, _CH, F), _f32),
            pltpu.SemaphoreType.DMA((_NBUF,)),
            pltpu.SemaphoreType.DMA((_NBUF,)),
        ],
    )
    def k(table_h, src_h, dst_h, zinit_h, out_h, spmem, idx_s, idx_d, rows,
          sem_g, sem_s):
        c = lax.axis_index("c")
        s = lax.axis_index("s")
        wid = s * _NC + c
        nps = NP // _NS
        pltpu.sync_copy(zinit_h.at[pl.ds(s * nps, nps)],
                        spmem.at[pl.ds(s * nps, nps)])
        pltpu.sync_copy(src_h.at[wid], idx_s)
        pltpu.sync_copy(dst_h.at[wid], idx_d)
        plsc.subcore_barrier()

        for b in range(_NBUF):
            pltpu.async_copy(table_h.at[idx_s.at[b]], rows.at[b], sem_g.at[b])

        def step(i, _):
            j0 = i * _NBUF
            for b in range(_NBUF):
                j = j0 + b
                pltpu.make_async_copy(table_h.at[idx_s.at[j]], rows.at[b],
                                      sem_g.at[b]).wait()
                pltpu.async_copy(rows.at[b], spmem.at[idx_d.at[j]],
                                 sem_s.at[b], add=True)

                @pl.when(j + _NBUF < K)
                def _():
                    pltpu.make_async_copy(rows.at[b], spmem.at[idx_d.at[j]],
                                          sem_s.at[b]).wait()
                    pltpu.async_copy(table_h.at[idx_s.at[j + _NBUF]],
                                     rows.at[b], sem_g.at[b])

            return ()

        lax.fori_loop(0, K // _NBUF, step, ())
        for b in range(_NBUF):
            pltpu.make_async_copy(rows.at[b],
                                  spmem.at[idx_d.at[K - _NBUF + b]],
                                  sem_s.at[b]).wait()
        plsc.subcore_barrier()
        nps = NP // _NS
        pltpu.sync_copy(spmem.at[pl.ds(s * nps, nps)],
                        out_h.at[c].at[pl.ds(s * nps, nps)])

    return k(table, src3, dst3, zinit)


def _tc_a(degp, x, w1):
    N = x.shape[0]
    H = w1.shape[1]

    def body(degp_ref, x_ref, w1_ref, hs1_ref, dinv_ref):
        deg = degp_ref[0, 0:N, 0:1] + degp_ref[1, 0:N, 0:1] + 1.0
        dinv = lax.rsqrt(deg)
        h = jnp.dot(x_ref[...], w1_ref[...], preferred_element_type=_f32)
        hs1_ref[...] = h * dinv
        dinv_ref[...] = dinv

    return pl.pallas_call(
        body,
        out_shape=(jax.ShapeDtypeStruct((N, H), _f32),
                   jax.ShapeDtypeStruct((N, 1), _f32)),
    )(degp, x, w1)


def _tc_b(m1, hs1, dinv, b1r, w2p):
    N = hs1.shape[0]
    F2 = w2p.shape[1]

    def body(m1_ref, hs1_ref, dinv_ref, b1_ref, w2_ref, hs2_ref):
        dinv = dinv_ref[...]
        z = dinv * (m1_ref[0, 0:N] + m1_ref[1, 0:N] + hs1_ref[...]) + b1_ref[...]
        z = jnp.maximum(z, 0.0)
        h2 = jnp.dot(z, w2_ref[...], preferred_element_type=_f32)
        hs2_ref[...] = h2 * dinv

    return pl.pallas_call(
        body,
        out_shape=jax.ShapeDtypeStruct((N, F2), _f32),
    )(m1, hs1, dinv, b1r, w2p)


def _tc_c(m2, hs2, dinv, b2r):
    N, F2 = hs2.shape

    def body(m2_ref, hs2_ref, dinv_ref, b2_ref, out_ref):
        out_ref[...] = (dinv_ref[...]
                        * (m2_ref[0, 0:N] + m2_ref[1, 0:N] + hs2_ref[...])
                        + b2_ref[...])

    return pl.pallas_call(
        body,
        out_shape=jax.ShapeDtypeStruct((N, F2), _f32),
    )(m2, hs2, dinv, b2r)


def kernel(x, edge_index, W1, b1, W2, b2):
    N, D = x.shape
    H = W1.shape[1]
    C = W2.shape[1]
    E = edge_index.shape[1]

    F2 = 48
    NP = -(-(N + 1) // 128) * 128
    per_w = -(-E // (_NW * 2 * _CH)) * (2 * _CH)
    K = per_w // _CH
    E_pad = per_w * _NW
    pad = E_pad - E

    src = edge_index[0]
    dst = edge_index[1]
    pad_src = jnp.arange(pad, dtype=jnp.int32) % N
    pad_dst = N + jnp.arange(pad, dtype=jnp.int32) % (NP - N)
    src3 = jnp.concatenate([src, pad_src]).reshape(_NW, K, _CH)
    dst3 = jnp.concatenate([dst, pad_dst]).reshape(_NW, K, _CH)
    zeros16 = jnp.zeros((NP, H), _f32)
    zeros48 = jnp.zeros((NP, F2), _f32)
    ones8 = jnp.ones((_CH, 8), _f32)
    zeros8 = jnp.zeros((NP, 8), _f32)
    w2p = jnp.pad(W2, ((0, 0), (0, F2 - C)))
    b1r = b1.reshape(1, H)
    b2r = jnp.pad(b2, (0, F2 - C)).reshape(1, F2)

    degp = _deg_call(dst3, ones8, zeros8, NP=NP, K=K)
    hs1, dinv = _tc_a(degp, x, W1)
    m1 = _agg_call(hs1, src3, dst3, zeros16, NP=NP, K=K)
    hs2 = _tc_b(m1, hs1, dinv, b1r, w2p)
    m2 = _agg_call(hs2, src3, dst3, zeros48, NP=NP, K=K)
    out48 = _tc_c(m2, hs2, dinv, b2r)
    return out48[:, :C]


# speedup vs baseline: 55.9703x; 1.0238x over previous
import functools

import jax
import jax.numpy as jnp
from jax import lax
from jax.experimental import pallas as pl
from jax.experimental.pallas import tpu as pltpu
from jax.experimental.pallas import tpu_sc as plsc

_NC = 2
_NS = 16
_NW = _NC * _NS
_CH = 128
_NBUF = 8

_f32 = jnp.float32


def _mesh():
    return plsc.VectorSubcoreMesh(core_axis_name="c", subcore_axis_name="s")


def _deg_call(dst3, ones_rows, zinit, *, NP, K):
    F = ones_rows.shape[1]

    @functools.partial(
        pl.kernel,
        mesh=_mesh(),
        compiler_params=pltpu.CompilerParams(use_tc_tiling_on_sc=False),
        out_type=jax.ShapeDtypeStruct((_NC, NP, F), _f32),
        scratch_types=[
            pltpu.VMEM_SHARED((NP, F), _f32),
            pltpu.VMEM((K, _CH), jnp.int32),
            pltpu.VMEM((_CH, F), _f32),
            pltpu.SemaphoreType.DMA((_NBUF,)),
        ],
    )
    def k(dst_h, ones_h, zinit_h, out_h, spmem, idx_d, rows, sem_s):
        c = lax.axis_index("c")
        s = lax.axis_index("s")
        wid = s * _NC + c
        nps = NP // _NS
        pltpu.sync_copy(zinit_h.at[pl.ds(s * nps, nps)],
                        spmem.at[pl.ds(s * nps, nps)])
        pltpu.sync_copy(dst_h.at[wid], idx_d)
        pltpu.sync_copy(ones_h, rows)
        plsc.subcore_barrier()

        for b in range(_NBUF):
            pltpu.async_copy(rows, spmem.at[idx_d.at[b]], sem_s.at[b],
                             add=True)

        def step(i, _):
            j0 = i * _NBUF
            for b in range(_NBUF):
                j = j0 + b

                @pl.when(j + _NBUF < K)
                def _():
                    pltpu.make_async_copy(rows, spmem.at[idx_d.at[j]],
                                          sem_s.at[b]).wait()
                    pltpu.async_copy(rows, spmem.at[idx_d.at[j + _NBUF]],
                                     sem_s.at[b], add=True)

            return ()

        lax.fori_loop(0, K // _NBUF, step, ())
        for b in range(_NBUF):
            pltpu.make_async_copy(rows, spmem.at[idx_d.at[K - _NBUF + b]],
                                  sem_s.at[b]).wait()
        plsc.subcore_barrier()
        nps = NP // _NS
        pltpu.sync_copy(spmem.at[pl.ds(s * nps, nps)],
                        out_h.at[c].at[pl.ds(s * nps, nps)])

    return k(dst3, ones_rows, zinit)


def _agg_call(table, src3, dst3, zinit, *, NP, K):
    F = table.shape[1]

    @functools.partial(
        pl.kernel,
        mesh=_mesh(),
        compiler_params=pltpu.CompilerParams(use_tc_tiling_on_sc=False),
        out_type=jax.ShapeDtypeStruct((_NC, NP, F), _f32),
        scratch_types=[
            pltpu.VMEM_SHARED((NP, F), _f32),
            pltpu.VMEM((K, _CH), jnp.int32),
            pltpu.VMEM((K, _CH), jnp.int32),
            pltpu.VMEM((_NBUF, _CH, F), _f32),
            pltpu.SemaphoreType.DMA((_NBUF,)),
            pltpu.SemaphoreType.DMA((_NBUF,)),
        ],
    )
    def k(table_h, src_h, dst_h, zinit_h, out_h, spmem, idx_s, idx_d, rows,
          sem_g, sem_s):
        c = lax.axis_index("c")
        s = lax.axis_index("s")
        wid = s * _NC + c
        nps = NP // _NS
        pltpu.sync_copy(zinit_h.at[pl.ds(s * nps, nps)],
                        spmem.at[pl.ds(s * nps, nps)])
        pltpu.sync_copy(src_h.at[wid], idx_s)
        pltpu.sync_copy(dst_h.at[wid], idx_d)
        plsc.subcore_barrier()

        for b in range(_NBUF):
            pltpu.async_copy(table_h.at[idx_s.at[b]], rows.at[b], sem_g.at[b])

        def step(i, _):
            j0 = i * _NBUF
            for b in range(_NBUF):
                j = j0 + b
                pltpu.make_async_copy(table_h.at[idx_s.at[j]], rows.at[b],
                                      sem_g.at[b]).wait()
                pltpu.async_copy(rows.at[b], spmem.at[idx_d.at[j]],
                                 sem_s.at[b], add=True)

                @pl.when(j + _NBUF < K)
                def _():
                    pltpu.make_async_copy(rows.at[b], spmem.at[idx_d.at[j]],
                                          sem_s.at[b]).wait()
                    pltpu.async_copy(table_h.at[idx_s.at[j + _NBUF]],
                                     rows.at[b], sem_g.at[b])

            return ()

        lax.fori_loop(0, K // _NBUF, step, ())
        for b in range(_NBUF):
            pltpu.make_async_copy(rows.at[b],
                                  spmem.at[idx_d.at[K - _NBUF + b]],
                                  sem_s.at[b]).wait()
        plsc.subcore_barrier()
        nps = NP // _NS
        pltpu.sync_copy(spmem.at[pl.ds(s * nps, nps)],
                        out_h.at[c].at[pl.ds(s * nps, nps)])

    return k(table, src3, dst3, zinit)


def _tc_a(degp, x, w1):
    N = x.shape[0]
    H = w1.shape[1]

    def body(degp_ref, x_ref, w1_ref, hs1_ref, dinv_ref):
        deg = degp_ref[0, 0:N, 0:1] + degp_ref[1, 0:N, 0:1] + 1.0
        dinv = lax.rsqrt(deg)
        h = jnp.dot(x_ref[...], w1_ref[...], preferred_element_type=_f32)
        hs1_ref[...] = h * dinv
        dinv_ref[...] = dinv

    return pl.pallas_call(
        body,
        out_shape=(jax.ShapeDtypeStruct((N, H), _f32),
                   jax.ShapeDtypeStruct((N, 1), _f32)),
    )(degp, x, w1)


def _tc_b(m1, hs1, dinv, b1r, w2p):
    N = hs1.shape[0]
    F2 = w2p.shape[1]

    def body(m1_ref, hs1_ref, dinv_ref, b1_ref, w2_ref, hs2_ref):
        dinv = dinv_ref[...]
        z = dinv * (m1_ref[0, 0:N] + m1_ref[1, 0:N] + hs1_ref[...]) + b1_ref[...]
        z = jnp.maximum(z, 0.0)
        h2 = jnp.dot(z, w2_ref[...], preferred_element_type=_f32)
        hs2_ref[...] = h2 * dinv

    return pl.pallas_call(
        body,
        out_shape=jax.ShapeDtypeStruct((N, F2), _f32),
    )(m1, hs1, dinv, b1r, w2p)


def _tc_c(m2, hs2, dinv, b2r):
    N, F2 = hs2.shape

    def body(m2_ref, hs2_ref, dinv_ref, b2_ref, out_ref):
        out_ref[...] = (dinv_ref[...]
                        * (m2_ref[0, 0:N] + m2_ref[1, 0:N] + hs2_ref[...])
                        + b2_ref[...])

    return pl.pallas_call(
        body,
        out_shape=jax.ShapeDtypeStruct((N, F2), _f32),
    )(m2, hs2, dinv, b2r)


def kernel(x, edge_index, W1, b1, W2, b2):
    N, D = x.shape
    H = W1.shape[1]
    C = W2.shape[1]
    E = edge_index.shape[1]

    F2 = 48
    NP = -(-(N + 1) // 128) * 128
    per_w = -(-E // (_NW * 2 * _CH)) * (2 * _CH)
    K = per_w // _CH
    E_pad = per_w * _NW
    pad = E_pad - E

    src = edge_index[0]
    dst = edge_index[1]
    pad_src = jnp.arange(pad, dtype=jnp.int32) % N
    pad_dst = N + jnp.arange(pad, dtype=jnp.int32) % (NP - N)
    src3 = jnp.concatenate([src, pad_src]).reshape(_NW, K, _CH)
    dst3 = jnp.concatenate([dst, pad_dst]).reshape(_NW, K, _CH)
    zeros16 = jnp.zeros((NP, H), _f32)
    zeros48 = jnp.zeros((NP, F2), _f32)
    ones8 = jnp.ones((_CH, 8), _f32)
    zeros8 = jnp.zeros((NP, 8), _f32)
    w2p = jnp.pad(W2, ((0, 0), (0, F2 - C)))
    b1r = b1.reshape(1, H)
    b2r = jnp.pad(b2, (0, F2 - C)).reshape(1, F2)

    degp = _deg_call(dst3, ones8, zeros8, NP=NP, K=K)
    hs1, dinv = _tc_a(degp, x, W1)
    m1 = _agg_call(hs1, src3, dst3, zeros16, NP=NP, K=K)
    hs2 = _tc_b(m1, hs1, dinv, b1r, w2p)
    m2 = _agg_call(hs2, src3, dst3, zeros48, NP=NP, K=K)
    out48 = _tc_c(m2, hs2, dinv, b2r)
    return out48[:, :C]


# speedup vs baseline: 57.4277x; 1.0260x over previous
import functools

import jax
import jax.numpy as jnp
from jax import lax
from jax.experimental import pallas as pl
from jax.experimental.pallas import tpu as pltpu
from jax.experimental.pallas import tpu_sc as plsc

_NC = 2
_NS = 16
_NW = _NC * _NS
_CH = 128
_NBUF = 8

_f32 = jnp.float32


def _mesh():
    return plsc.VectorSubcoreMesh(core_axis_name="c", subcore_axis_name="s")


def _deg_call(dst3, ones_rows, zinit, *, NP, K):
    F = ones_rows.shape[1]

    @functools.partial(
        pl.kernel,
        mesh=_mesh(),
        compiler_params=pltpu.CompilerParams(use_tc_tiling_on_sc=False),
        out_type=jax.ShapeDtypeStruct((_NC, NP, F), _f32),
        scratch_types=[
            pltpu.VMEM_SHARED((NP, F), _f32),
            pltpu.VMEM((K, _CH), jnp.int32),
            pltpu.VMEM((_CH, F), _f32),
            pltpu.SemaphoreType.DMA((_NBUF,)),
        ],
    )
    def k(dst_h, ones_h, zinit_h, out_h, spmem, idx_d, rows, sem_s):
        c = lax.axis_index("c")
        s = lax.axis_index("s")
        wid = s * _NC + c
        nps = NP // _NS
        pltpu.sync_copy(zinit_h.at[pl.ds(s * nps, nps)],
                        spmem.at[pl.ds(s * nps, nps)])
        pltpu.sync_copy(dst_h.at[wid], idx_d)
        pltpu.sync_copy(ones_h, rows)
        plsc.subcore_barrier()

        for b in range(_NBUF):
            pltpu.async_copy(rows, spmem.at[idx_d.at[b]], sem_s.at[b],
                             add=True)

        def step(i, _):
            j0 = i * _NBUF
            for b in range(_NBUF):
                j = j0 + b

                @pl.when(j + _NBUF < K)
                def _():
                    pltpu.make_async_copy(rows, spmem.at[idx_d.at[j]],
                                          sem_s.at[b]).wait()
                    pltpu.async_copy(rows, spmem.at[idx_d.at[j + _NBUF]],
                                     sem_s.at[b], add=True)

            return ()

        lax.fori_loop(0, K // _NBUF, step, ())
        for b in range(_NBUF):
            pltpu.make_async_copy(rows, spmem.at[idx_d.at[K - _NBUF + b]],
                                  sem_s.at[b]).wait()
        plsc.subcore_barrier()
        nps = NP // _NS
        pltpu.sync_copy(spmem.at[pl.ds(s * nps, nps)],
                        out_h.at[c].at[pl.ds(s * nps, nps)])

    return k(dst3, ones_rows, zinit)


def _agg_call(table, src3, dst3, zinit, *, NP, K):
    F = table.shape[1]

    @functools.partial(
        pl.kernel,
        mesh=_mesh(),
        compiler_params=pltpu.CompilerParams(use_tc_tiling_on_sc=False),
        out_type=jax.ShapeDtypeStruct((_NC, NP, F), _f32),
        scratch_types=[
            pltpu.VMEM_SHARED((NP, F), _f32),
            pltpu.VMEM((K, _CH), jnp.int32),
            pltpu.VMEM((K, _CH), jnp.int32),
            pltpu.VMEM((_NBUF, _CH, F), _f32),
            pltpu.SemaphoreType.DMA((_NBUF,)),
            pltpu.SemaphoreType.DMA((_NBUF,)),
        ],
    )
    def k(table_h, src_h, dst_h, zinit_h, out_h, spmem, idx_s, idx_d, rows,
          sem_g, sem_s):
        c = lax.axis_index("c")
        s = lax.axis_index("s")
        wid = s * _NC + c
        nps = NP // _NS
        pltpu.sync_copy(zinit_h.at[pl.ds(s * nps, nps)],
                        spmem.at[pl.ds(s * nps, nps)])
        pltpu.sync_copy(src_h.at[wid], idx_s)
        pltpu.sync_copy(dst_h.at[wid], idx_d)
        plsc.subcore_barrier()

        for b in range(_NBUF):
            pltpu.async_copy(table_h.at[idx_s.at[b]], rows.at[b], sem_g.at[b])

        def step(i, _):
            j0 = i * _NBUF
            for b in range(_NBUF):
                j = j0 + b
                pltpu.make_async_copy(table_h.at[idx_s.at[j]], rows.at[b],
                                      sem_g.at[b]).wait()
                pltpu.async_copy(rows.at[b], spmem.at[idx_d.at[j]],
                                 sem_s.at[b], add=True)

                @pl.when(j + _NBUF < K)
                def _():
                    pltpu.make_async_copy(rows.at[b], spmem.at[idx_d.at[j]],
                                          sem_s.at[b]).wait()
                    pltpu.async_copy(table_h.at[idx_s.at[j + _NBUF]],
                                     rows.at[b], sem_g.at[b])

            return ()

        lax.fori_loop(0, K // _NBUF, step, ())
        for b in range(_NBUF):
            pltpu.make_async_copy(rows.at[b],
                                  spmem.at[idx_d.at[K - _NBUF + b]],
                                  sem_s.at[b]).wait()
        plsc.subcore_barrier()
        nps = NP // _NS
        pltpu.sync_copy(spmem.at[pl.ds(s * nps, nps)],
                        out_h.at[c].at[pl.ds(s * nps, nps)])

    return k(table, src3, dst3, zinit)


def _tc_a(degp, x, w1):
    N = x.shape[0]
    H = w1.shape[1]

    def body(degp_ref, x_ref, w1_ref, hs1_ref, dinv_ref):
        deg = degp_ref[0, 0:N, 0:1] + degp_ref[1, 0:N, 0:1] + 1.0
        dinv = lax.rsqrt(deg)
        h = jnp.dot(x_ref[...], w1_ref[...], preferred_element_type=_f32)
        hs1_ref[...] = h * dinv
        dinv_ref[...] = dinv

    return pl.pallas_call(
        body,
        out_shape=(jax.ShapeDtypeStruct((N, H), _f32),
                   jax.ShapeDtypeStruct((N, 1), _f32)),
    )(degp, x, w1)


def _tc_b(m1, hs1, dinv, b1r, w2p):
    N = hs1.shape[0]
    F2 = w2p.shape[1]

    def body(m1_ref, hs1_ref, dinv_ref, b1_ref, w2_ref, hs2_ref):
        dinv = dinv_ref[...]
        z = dinv * (m1_ref[0, 0:N] + m1_ref[1, 0:N] + hs1_ref[...]) + b1_ref[...]
        z = jnp.maximum(z, 0.0)
        h2 = jnp.dot(z, w2_ref[...], preferred_element_type=_f32)
        hs2_ref[...] = h2 * dinv

    return pl.pallas_call(
        body,
        out_shape=jax.ShapeDtypeStruct((N, F2), _f32),
    )(m1, hs1, dinv, b1r, w2p)


def _tc_c(m2, hs2, dinv, b2r):
    N, F2 = hs2.shape

    def body(m2_ref, hs2_ref, dinv_ref, b2_ref, out_ref):
        out_ref[...] = (dinv_ref[...]
                        * (m2_ref[0, 0:N] + m2_ref[1, 0:N] + hs2_ref[...])
                        + b2_ref[...])

    return pl.pallas_call(
        body,
        out_shape=jax.ShapeDtypeStruct((N, F2), _f32),
    )(m2, hs2, dinv, b2r)


def kernel(x, edge_index, W1, b1, W2, b2):
    N, D = x.shape
    H = W1.shape[1]
    C = W2.shape[1]
    E = edge_index.shape[1]

    F2 = C
    NP = -(-(N + 1) // 128) * 128
    per_w = -(-E // (_NW * 2 * _CH)) * (2 * _CH)
    K = per_w // _CH
    E_pad = per_w * _NW
    pad = E_pad - E

    src = edge_index[0]
    dst = edge_index[1]
    pad_src = jnp.arange(pad, dtype=jnp.int32) % N
    pad_dst = N + jnp.arange(pad, dtype=jnp.int32) % (NP - N)
    src3 = jnp.concatenate([src, pad_src]).reshape(_NW, K, _CH)
    dst3 = jnp.concatenate([dst, pad_dst]).reshape(_NW, K, _CH)
    zeros16 = jnp.zeros((NP, H), _f32)
    zeros48 = jnp.zeros((NP, F2), _f32)
    ones8 = jnp.ones((_CH, 8), _f32)
    zeros8 = jnp.zeros((NP, 8), _f32)
    w2p = jnp.pad(W2, ((0, 0), (0, F2 - C)))
    b1r = b1.reshape(1, H)
    b2r = jnp.pad(b2, (0, F2 - C)).reshape(1, F2)

    degp = _deg_call(dst3, ones8, zeros8, NP=NP, K=K)
    hs1, dinv = _tc_a(degp, x, W1)
    m1 = _agg_call(hs1, src3, dst3, zeros16, NP=NP, K=K)
    hs2 = _tc_b(m1, hs1, dinv, b1r, w2p)
    m2 = _agg_call(hs2, src3, dst3, zeros48, NP=NP, K=K)
    out48 = _tc_c(m2, hs2, dinv, b2r)
    return out48[:, :C]
